# baseline (speedup 1.0000x reference)
import functools
import math

import jax
import jax.numpy as jnp
from jax import lax
from jax.experimental import pallas as pl
from jax.experimental.pallas import tpu as pltpu
from jax.experimental.pallas import tpu_sc as plsc

B, N, NF, CLS, K = 4, 4096, 128, 16, 16
F2 = NF * 2
BN = B * N
FILM_K = 0.5
BN_SCALE = 1.0 / math.sqrt(1.0 + 1e-5)

NC, NS = 2, 16
NW = NC * NS
NPW = BN // NW

KNN_R = 256
KNN_L = 5
KNN_NBLK = 128
KNN_NPOS = N // KNN_NBLK
INV_K = 1.0 / K


def _knn_body(ptsT_ref, x_ref, featT_ref, w1, b1, w2, b2, w3, b3,
              idx_ref, h_ref):
    rows = ptsT_ref[0]
    cols = x_ref[0]
    sqc = jnp.sum(cols * cols, axis=0, keepdims=True)
    sqr = jnp.sum(rows * rows, axis=1, keepdims=True)
    dot = jnp.dot(rows, cols, preferred_element_type=jnp.float32)
    d = sqr + sqc - 2.0 * dot
    orig = [d[:, s * KNN_NBLK:(s + 1) * KNN_NBLK] for s in range(KNN_NPOS)]
    cur = list(orig)
    lane = lax.broadcasted_iota(jnp.int32, (KNN_R, KNN_NBLK), 1)
    vs, gs = [], []
    for l in range(KNN_L):
        bm = cur[0]
        for s in range(1, KNN_NPOS):
            bm = jnp.minimum(bm, cur[s])
        ap = jnp.full((KNN_R, KNN_NBLK), KNN_NPOS, jnp.int32)
        for s in reversed(range(KNN_NPOS)):
            ap = jnp.where(cur[s] == bm, jnp.int32(s), ap)
        vs.append(bm)
        gs.append(ap * KNN_NBLK + lane)
        if l < KNN_L - 1:
            for s in range(KNN_NPOS):
                cur[s] = jnp.where(ap == jnp.int32(s), jnp.float32(jnp.inf), cur[s])
    bmv, rep = vs[0], gs[0]
    lvl = jnp.zeros((KNN_R, KNN_NBLK), jnp.int32)
    INF = jnp.full((KNN_R, KNN_NBLK), jnp.inf, jnp.float32)
    NEG = jnp.full((KNN_R, KNN_NBLK), -1, jnp.int32)
    outs = []
    m = a = None
    for k in range(K):
        m = jnp.min(bmv, axis=1, keepdims=True)
        a = jnp.min(jnp.where(bmv == m, rep, jnp.int32(N)), axis=1, keepdims=True)
        outs.append(a)
        if k < K - 1:
            w = rep == a
            lvl = lvl + w.astype(jnp.int32)
            nv, ng = INF, NEG
            for l in range(KNN_L - 1, 0, -1):
                c = lvl == l
                nv = jnp.where(c, vs[l], nv)
                ng = jnp.where(c, gs[l], ng)
            bmv = jnp.where(w, nv, bmv)
            rep = jnp.where(w, ng, rep)
    idx_ref[0] = jnp.concatenate(outs, axis=1)

    Ms = []
    for s in range(KNN_NPOS):
        gid = lane + jnp.int32(s * KNN_NBLK)
        cond = (orig[s] < m) | ((orig[s] == m) & (gid <= a))
        Ms.append(cond.astype(jnp.float32))
    M = jnp.concatenate(Ms, axis=1)
    S = jnp.dot(M, featT_ref[0], preferred_element_type=jnp.float32)
    Sx, Sy, Sz = S[:, 0:1], S[:, 1:2], S[:, 2:3]
    c00 = S[:, 3:4] - Sx * Sx * INV_K
    c01 = S[:, 4:5] - Sx * Sy * INV_K
    c02 = S[:, 5:6] - Sx * Sz * INV_K
    c11 = S[:, 6:7] - Sy * Sy * INV_K
    c12 = S[:, 7:8] - Sy * Sz * INV_K
    c22 = S[:, 8:9] - Sz * Sz * INV_K
    inp = jnp.concatenate(
        [rows, c00, c01, c02, c01, c11, c12, c02, c12, c22], axis=1)
    h = jnp.maximum(
        jnp.dot(inp, w1[...], preferred_element_type=jnp.float32) + b1[...], 0.0)
    h = jnp.maximum(
        jnp.dot(h, w2[...], preferred_element_type=jnp.float32) + b2[...], 0.0)
    h = jnp.maximum(
        jnp.dot(h, w3[...], preferred_element_type=jnp.float32) + b3[...], 0.0)
    h_ref[...] = h


def _knn_cov_mlp1(ptsT, x, featT, w1, b1, w2, b2, w3, b3):
    full = lambda *s: pl.BlockSpec(s, lambda b, t: tuple(0 for _ in s))
    return pl.pallas_call(
        _knn_body,
        grid=(B, N // KNN_R),
        in_specs=[
            pl.BlockSpec((1, KNN_R, 3), lambda b, t: (b, t, 0)),
            pl.BlockSpec((1, 3, N), lambda b, t: (b, 0, 0)),
            pl.BlockSpec((1, N, 9), lambda b, t: (b, 0, 0)),
            full(12, 64), full(1, 64), full(64, 64), full(1, 64),
            full(64, 64), full(1, 64),
        ],
        out_specs=[
            pl.BlockSpec((1, KNN_R, K), lambda b, t: (b, t, 0)),
            pl.BlockSpec((KNN_R, 64), lambda b, t: (b * (N // KNN_R) + t, 0)),
        ],
        out_shape=[
            jax.ShapeDtypeStruct((B, N, K), jnp.int32),
            jax.ShapeDtypeStruct((BN, 64), jnp.float32),
        ],
    )(ptsT, x, featT, w1, b1, w2, b2, w3, b3)


AGG_CH = 8
AGG_NCHK = NPW // AGG_CH
AGG_NBUF = 2


def _agg_kernel_body(h, colg2, out, idxb, rb0, rb1, outb, sem0, sem1):
    wid = lax.axis_index("s") * NC + lax.axis_index("c")
    base = wid * NPW
    pltpu.sync_copy(colg2.at[pl.ds(wid * AGG_NCHK, AGG_NCHK)], idxb)
    rbs = (rb0, rb1)
    sems = (sem0, sem1)

    def accum(rb, c):
        for n in range(AGG_CH):
            node = c * AGG_CH + n
            for q in range(4):
                acc = rb[n * K, pl.ds(q * 16, 16)]
                for r in range(1, K):
                    acc = acc + rb[n * K + r, pl.ds(q * 16, 16)]
                outb[node, pl.ds(q * 16, 16)] = acc

    for bnum in range(AGG_NBUF):
        pltpu.async_copy(h.at[idxb.at[bnum]], rbs[bnum], sems[bnum])

    def body(i, carry):
        c0 = i * AGG_NBUF
        for bnum in range(AGG_NBUF):
            c = c0 + bnum
            pltpu.make_async_copy(h.at[idxb.at[c]], rbs[bnum], sems[bnum]).wait()
            accum(rbs[bnum], c)

            @pl.when(c + AGG_NBUF < AGG_NCHK)
            def _():
                pltpu.async_copy(h.at[idxb.at[c + AGG_NBUF]], rbs[bnum], sems[bnum])

        return carry

    lax.fori_loop(0, AGG_NCHK // AGG_NBUF, body, 0)
    pltpu.sync_copy(outb, out.at[pl.ds(base, NPW)])


_agg_call = functools.partial(
    pl.kernel,
    out_type=jax.ShapeDtypeStruct((BN, 64), jnp.float32),
    mesh=plsc.VectorSubcoreMesh(
        core_axis_name="c", subcore_axis_name="s", num_cores=NC, num_subcores=NS
    ),
    scratch_types=[
        pltpu.VMEM((AGG_NCHK, AGG_CH * K), jnp.int32),
        pltpu.VMEM((AGG_CH * K, 64), jnp.float32),
        pltpu.VMEM((AGG_CH * K, 64), jnp.float32),
        pltpu.VMEM((NPW, 64), jnp.float32),
        pltpu.SemaphoreType.DMA,
        pltpu.SemaphoreType.DMA,
    ],
    compiler_params=pltpu.CompilerParams(
        needs_layout_passes=False, use_tc_tiling_on_sc=False
    ),
)(_agg_kernel_body)


D2_R = 1024
D2_TPB = N // D2_R
D2_STEPS = BN // D2_R


def _film_body(agg_ref, pe_ref, wg, bg, pw1, pb1, pw2, pb2,
               w2a, b2a, w2b, b2b, clss_ref, mcw, mcb, gaw, gab,
               bew, beb, fcw, fcb, out_ref, pool_s):
    t = pl.program_id(0)
    b = t // D2_TPB
    g = jnp.dot(agg_ref[...], wg[...], preferred_element_type=jnp.float32) + bg[...]
    f1 = jnp.maximum(
        jnp.dot(g, pw1[...], preferred_element_type=jnp.float32) + pb1[...], 0.0)
    film = jnp.dot(f1, pw2[...], preferred_element_type=jnp.float32) + pb2[...]
    gm = film[:, :F2]
    bt = film[:, F2:]
    h2 = gm * pe_ref[...] + bt
    h2 = jnp.maximum(
        jnp.dot(h2, w2a[...], preferred_element_type=jnp.float32) + b2a[...], 0.0)
    h2 = jnp.maximum(
        jnp.dot(h2, w2b[...], preferred_element_type=jnp.float32) + b2b[...], 0.0)
    tmax = jnp.max(h2, axis=0, keepdims=True)

    @pl.when(t % D2_TPB == 0)
    def _():
        pool_s[pl.ds(b, 1), :] = tmax

    @pl.when(t % D2_TPB != 0)
    def _():
        pool_s[pl.ds(b, 1), :] = jnp.maximum(pool_s[pl.ds(b, 1), :], tmax)

    @pl.when(t == D2_STEPS - 1)
    def _():
        xx = pool_s[...]
        ce = jnp.maximum(
            jnp.dot(clss_ref[...], mcw[...], preferred_element_type=jnp.float32)
            + mcb[...], 0.0)
        ga = jnp.maximum(
            jnp.dot(ce, gaw[...], preferred_element_type=jnp.float32) + gab[...], 0.0)
        be = jnp.maximum(
            jnp.dot(ce, bew[...], preferred_element_type=jnp.float32) + beb[...], 0.0)
        xx = FILM_K * (ga * xx + be) + (1.0 - FILM_K) * xx
        o = jnp.dot(xx, fcw[...], preferred_element_type=jnp.float32) + fcb[...]
        nrm = jnp.maximum(jnp.sqrt(jnp.sum(o * o, axis=1, keepdims=True)), 1e-12)
        out_ref[...] = o / nrm


def _film_mlp2_pool_head(aggsum, peT, wg, bg, pw1, pb1, pw2, pb2,
                         w2a, b2a, w2b, b2b, clssm, mcw, mcb, gaw, gab,
                         bew, beb, fcw, fcb):
    full = lambda *s: pl.BlockSpec(s, lambda t: tuple(0 for _ in s))
    return pl.pallas_call(
        _film_body,
        grid=(D2_STEPS,),
        in_specs=[
            pl.BlockSpec((D2_R, 64), lambda t: (t, 0)),
            pl.BlockSpec((D2_R, F2), lambda t: (t % D2_TPB, 0)),
            full(64, F2), full(1, F2),
            full(F2, NF), full(1, NF),
            full(NF, 2 * F2), full(1, 2 * F2),
            full(F2, F2), full(1, F2),
            full(F2, F2), full(1, F2),
            full(B, CLS), full(CLS, NF), full(1, NF),
            full(NF, F2), full(1, F2),
            full(NF, F2), full(1, F2),
            full(F2, NF), full(1, NF),
        ],
        out_specs=pl.BlockSpec((B, NF), lambda t: (0, 0)),
        out_shape=jax.ShapeDtypeStruct((B, NF), jnp.float32),
        scratch_shapes=[pltpu.VMEM((B, F2), jnp.float32)],
    )(aggsum, peT, wg, bg, pw1, pb1, pw2, pb2, w2a, b2a, w2b, b2b,
      clssm, mcw, mcb, gaw, gab, bew, beb, fcw, fcb)


def _fold(w, bc, g, be):
    s = g * BN_SCALE
    return (w * s[:, None]).T, (s * bc + be)[None, :]


def kernel(x, clss, mask, params, buffers):
    p = params
    ptsT = jnp.transpose(x, (0, 2, 1))
    px, py, pz = ptsT[..., 0], ptsT[..., 1], ptsT[..., 2]
    featT = jnp.stack(
        [px, py, pz, px * px, px * py, px * pz, py * py, py * pz, pz * pz],
        axis=-1)

    w1, b1 = _fold(p['mlp1_w1'], p['mlp1_b1'], p['mlp1_g1'], p['mlp1_be1'])
    w2, b2 = _fold(p['mlp1_w2'], p['mlp1_b2'], p['mlp1_g2'], p['mlp1_be2'])
    w3, b3 = _fold(p['mlp1_w3'], p['mlp1_b3'], p['mlp1_g3'], p['mlp1_be3'])
    idx, h = _knn_cov_mlp1(ptsT, x, featT, w1, b1, w2, b2, w3, b3)

    col = (idx + (jnp.arange(B, dtype=jnp.int32) * N)[:, None, None]).reshape(-1)
    row = jnp.repeat(jnp.arange(BN, dtype=jnp.int32), K)
    gl_idx = jnp.stack([row, col])

    colg2 = col.reshape(BN * K // 128, 128)
    aggsum = _agg_call(h, colg2)

    wg = p['g1_w'].T * INV_K
    bg = p['g1_b'][None, :]
    pw1, pb1 = p['pe_w1'].T, p['pe_b1'][None, :]
    pw2, pb2 = p['pe_w2'].T, p['pe_b2'][None, :]
    w2a, b2a = _fold(p['mlp2_w1'], p['mlp2_b1'], p['mlp2_g1'], p['mlp2_be1'])
    w2b, b2b = _fold(p['mlp2_w2'], p['mlp2_b2'], p['mlp2_g2'], p['mlp2_be2'])
    peT = jnp.transpose(buffers['pe'][0, :, :N], (1, 0))
    mcw, mcb = _fold(p['mc_w'], p['mc_b'], p['mc_g'], p['mc_be'])
    gaw, gab = _fold(p['gam_w'], p['gam_b'], p['gam_g'], p['gam_be'])
    bew, beb = _fold(p['bet_w'], p['bet_b'], p['bet_g'], p['bet_be'])
    fcw, fcb = p['fc_w'].T, p['fc_b'][None, :]
    clssm = clss[:, :, 0]
    out = _film_mlp2_pool_head(aggsum, peT, wg, bg, pw1, pb1, pw2, pb2,
                               w2a, b2a, w2b, b2b, clssm, mcw, mcb,
                               gaw, gab, bew, beb, fcw, fcb)
    return out, gl_idx


# speedup vs baseline: 1.1470x; 1.1470x over previous
import functools
import math

import jax
import jax.numpy as jnp
from jax import lax
from jax.experimental import pallas as pl
from jax.experimental.pallas import tpu as pltpu
from jax.experimental.pallas import tpu_sc as plsc

B, N, NF, CLS, K = 4, 4096, 128, 16, 16
F2 = NF * 2
BN = B * N
FILM_K = 0.5
BN_SCALE = 1.0 / math.sqrt(1.0 + 1e-5)

NC, NS = 2, 16
NW = NC * NS
NPW = BN // NW

KNN_R = 512
KNN_L = 5
KNN_NBLK = 128
KNN_NPOS = N // KNN_NBLK
INV_K = 1.0 / K


def _knn_body(ptsT_ref, x_ref, featT_ref, w1, b1, w2, b2, w3, b3,
              idx_ref, h_ref):
    rows = ptsT_ref[0]
    cols = x_ref[0]
    sqc = jnp.sum(cols * cols, axis=0, keepdims=True)
    sqr = jnp.sum(rows * rows, axis=1, keepdims=True)
    dot = jnp.dot(rows, cols, preferred_element_type=jnp.float32)
    d = sqr + sqc - 2.0 * dot
    orig = [d[:, s * KNN_NBLK:(s + 1) * KNN_NBLK] for s in range(KNN_NPOS)]
    cur = list(orig)
    lane = lax.broadcasted_iota(jnp.int32, (KNN_R, KNN_NBLK), 1)
    vs, gs = [], []
    for l in range(KNN_L):
        bm = cur[0]
        for s in range(1, KNN_NPOS):
            bm = jnp.minimum(bm, cur[s])
        ap = jnp.full((KNN_R, KNN_NBLK), KNN_NPOS, jnp.int32)
        for s in reversed(range(KNN_NPOS)):
            ap = jnp.where(cur[s] == bm, jnp.int32(s), ap)
        vs.append(bm)
        gs.append(ap * KNN_NBLK + lane)
        if l < KNN_L - 1:
            for s in range(KNN_NPOS):
                cur[s] = jnp.where(ap == jnp.int32(s), jnp.float32(jnp.inf), cur[s])
    bmv, rep = vs[0], gs[0]
    lvl = jnp.zeros((KNN_R, KNN_NBLK), jnp.int32)
    INF = jnp.full((KNN_R, KNN_NBLK), jnp.inf, jnp.float32)
    NEG = jnp.full((KNN_R, KNN_NBLK), -1, jnp.int32)
    outs = []
    m = a = None
    for k in range(K):
        m = jnp.min(bmv, axis=1, keepdims=True)
        a = jnp.min(jnp.where(bmv == m, rep, jnp.int32(N)), axis=1, keepdims=True)
        outs.append(a)
        if k < K - 1:
            w = rep == a
            lvl = lvl + w.astype(jnp.int32)
            nv, ng = INF, NEG
            for l in range(KNN_L - 1, 0, -1):
                c = lvl == l
                nv = jnp.where(c, vs[l], nv)
                ng = jnp.where(c, gs[l], ng)
            bmv = jnp.where(w, nv, bmv)
            rep = jnp.where(w, ng, rep)
    idx_ref[0] = jnp.concatenate(outs, axis=1)

    Ms = []
    for s in range(KNN_NPOS):
        gid = lane + jnp.int32(s * KNN_NBLK)
        cond = (orig[s] < m) | ((orig[s] == m) & (gid <= a))
        Ms.append(cond.astype(jnp.float32))
    M = jnp.concatenate(Ms, axis=1)
    S = jnp.dot(M, featT_ref[0], preferred_element_type=jnp.float32)
    Sx, Sy, Sz = S[:, 0:1], S[:, 1:2], S[:, 2:3]
    c00 = S[:, 3:4] - Sx * Sx * INV_K
    c01 = S[:, 4:5] - Sx * Sy * INV_K
    c02 = S[:, 5:6] - Sx * Sz * INV_K
    c11 = S[:, 6:7] - Sy * Sy * INV_K
    c12 = S[:, 7:8] - Sy * Sz * INV_K
    c22 = S[:, 8:9] - Sz * Sz * INV_K
    inp = jnp.concatenate(
        [rows, c00, c01, c02, c01, c11, c12, c02, c12, c22], axis=1)
    h = jnp.maximum(
        jnp.dot(inp, w1[...], preferred_element_type=jnp.float32) + b1[...], 0.0)
    h = jnp.maximum(
        jnp.dot(h, w2[...], preferred_element_type=jnp.float32) + b2[...], 0.0)
    h = jnp.maximum(
        jnp.dot(h, w3[...], preferred_element_type=jnp.float32) + b3[...], 0.0)
    h_ref[...] = h


def _knn_cov_mlp1(ptsT, x, featT, w1, b1, w2, b2, w3, b3):
    full = lambda *s: pl.BlockSpec(s, lambda b, t: tuple(0 for _ in s))
    return pl.pallas_call(
        _knn_body,
        grid=(B, N // KNN_R),
        in_specs=[
            pl.BlockSpec((1, KNN_R, 3), lambda b, t: (b, t, 0)),
            pl.BlockSpec((1, 3, N), lambda b, t: (b, 0, 0)),
            pl.BlockSpec((1, N, 9), lambda b, t: (b, 0, 0)),
            full(12, 64), full(1, 64), full(64, 64), full(1, 64),
            full(64, 64), full(1, 64),
        ],
        out_specs=[
            pl.BlockSpec((1, KNN_R, K), lambda b, t: (b, t, 0)),
            pl.BlockSpec((KNN_R, 64), lambda b, t: (b * (N // KNN_R) + t, 0)),
        ],
        out_shape=[
            jax.ShapeDtypeStruct((B, N, K), jnp.int32),
            jax.ShapeDtypeStruct((BN, 64), jnp.float32),
        ],
        compiler_params=pltpu.CompilerParams(
            dimension_semantics=("parallel", "parallel")),
    )(ptsT, x, featT, w1, b1, w2, b2, w3, b3)


AGG_CH = 8
AGG_NCHK = NPW // AGG_CH
AGG_NBUF = 2


def _agg_kernel_body(h, colg2, out, idxb, rb0, rb1, outb, sem0, sem1):
    wid = lax.axis_index("s") * NC + lax.axis_index("c")
    base = wid * NPW
    pltpu.sync_copy(colg2.at[pl.ds(wid * AGG_NCHK, AGG_NCHK)], idxb)
    rbs = (rb0, rb1)
    sems = (sem0, sem1)

    def accum(rb, c):
        for n in range(AGG_CH):
            node = c * AGG_CH + n
            for q in range(4):
                acc = rb[n * K, pl.ds(q * 16, 16)]
                for r in range(1, K):
                    acc = acc + rb[n * K + r, pl.ds(q * 16, 16)]
                outb[node, pl.ds(q * 16, 16)] = acc

    for bnum in range(AGG_NBUF):
        pltpu.async_copy(h.at[idxb.at[bnum]], rbs[bnum], sems[bnum])

    def body(i, carry):
        c0 = i * AGG_NBUF
        for bnum in range(AGG_NBUF):
            c = c0 + bnum
            pltpu.make_async_copy(h.at[idxb.at[c]], rbs[bnum], sems[bnum]).wait()
            accum(rbs[bnum], c)

            @pl.when(c + AGG_NBUF < AGG_NCHK)
            def _():
                pltpu.async_copy(h.at[idxb.at[c + AGG_NBUF]], rbs[bnum], sems[bnum])

        return carry

    lax.fori_loop(0, AGG_NCHK // AGG_NBUF, body, 0)
    pltpu.sync_copy(outb, out.at[pl.ds(base, NPW)])


_agg_call = functools.partial(
    pl.kernel,
    out_type=jax.ShapeDtypeStruct((BN, 64), jnp.float32),
    mesh=plsc.VectorSubcoreMesh(
        core_axis_name="c", subcore_axis_name="s", num_cores=NC, num_subcores=NS
    ),
    scratch_types=[
        pltpu.VMEM((AGG_NCHK, AGG_CH * K), jnp.int32),
        pltpu.VMEM((AGG_CH * K, 64), jnp.float32),
        pltpu.VMEM((AGG_CH * K, 64), jnp.float32),
        pltpu.VMEM((NPW, 64), jnp.float32),
        pltpu.SemaphoreType.DMA,
        pltpu.SemaphoreType.DMA,
    ],
    compiler_params=pltpu.CompilerParams(
        needs_layout_passes=False, use_tc_tiling_on_sc=False
    ),
)(_agg_kernel_body)


D2_R = 1024
D2_TPB = N // D2_R
D2_STEPS = BN // D2_R


def _film_body(agg_ref, pe_ref, wg, bg, pw1, pb1, pw2, pb2,
               w2a, b2a, w2b, b2b, clss_ref, mcw, mcb, gaw, gab,
               bew, beb, fcw, fcb, out_ref, pool_s):
    t = pl.program_id(0)
    b = t // D2_TPB
    g = jnp.dot(agg_ref[...], wg[...], preferred_element_type=jnp.float32) + bg[...]
    f1 = jnp.maximum(
        jnp.dot(g, pw1[...], preferred_element_type=jnp.float32) + pb1[...], 0.0)
    film = jnp.dot(f1, pw2[...], preferred_element_type=jnp.float32) + pb2[...]
    gm = film[:, :F2]
    bt = film[:, F2:]
    h2 = gm * pe_ref[...] + bt
    h2 = jnp.maximum(
        jnp.dot(h2, w2a[...], preferred_element_type=jnp.float32) + b2a[...], 0.0)
    h2 = jnp.maximum(
        jnp.dot(h2, w2b[...], preferred_element_type=jnp.float32) + b2b[...], 0.0)
    tmax = jnp.max(h2, axis=0, keepdims=True)

    @pl.when(t % D2_TPB == 0)
    def _():
        pool_s[pl.ds(b, 1), :] = tmax

    @pl.when(t % D2_TPB != 0)
    def _():
        pool_s[pl.ds(b, 1), :] = jnp.maximum(pool_s[pl.ds(b, 1), :], tmax)

    @pl.when(t == D2_STEPS - 1)
    def _():
        xx = pool_s[...]
        ce = jnp.maximum(
            jnp.dot(clss_ref[...], mcw[...], preferred_element_type=jnp.float32)
            + mcb[...], 0.0)
        ga = jnp.maximum(
            jnp.dot(ce, gaw[...], preferred_element_type=jnp.float32) + gab[...], 0.0)
        be = jnp.maximum(
            jnp.dot(ce, bew[...], preferred_element_type=jnp.float32) + beb[...], 0.0)
        xx = FILM_K * (ga * xx + be) + (1.0 - FILM_K) * xx
        o = jnp.dot(xx, fcw[...], preferred_element_type=jnp.float32) + fcb[...]
        nrm = jnp.maximum(jnp.sqrt(jnp.sum(o * o, axis=1, keepdims=True)), 1e-12)
        out_ref[...] = o / nrm


def _film_mlp2_pool_head(aggsum, peT, wg, bg, pw1, pb1, pw2, pb2,
                         w2a, b2a, w2b, b2b, clssm, mcw, mcb, gaw, gab,
                         bew, beb, fcw, fcb):
    full = lambda *s: pl.BlockSpec(s, lambda t: tuple(0 for _ in s))
    return pl.pallas_call(
        _film_body,
        grid=(D2_STEPS,),
        in_specs=[
            pl.BlockSpec((D2_R, 64), lambda t: (t, 0)),
            pl.BlockSpec((D2_R, F2), lambda t: (t % D2_TPB, 0)),
            full(64, F2), full(1, F2),
            full(F2, NF), full(1, NF),
            full(NF, 2 * F2), full(1, 2 * F2),
            full(F2, F2), full(1, F2),
            full(F2, F2), full(1, F2),
            full(B, CLS), full(CLS, NF), full(1, NF),
            full(NF, F2), full(1, F2),
            full(NF, F2), full(1, F2),
            full(F2, NF), full(1, NF),
        ],
        out_specs=pl.BlockSpec((B, NF), lambda t: (0, 0)),
        out_shape=jax.ShapeDtypeStruct((B, NF), jnp.float32),
        scratch_shapes=[pltpu.VMEM((B, F2), jnp.float32)],
    )(aggsum, peT, wg, bg, pw1, pb1, pw2, pb2, w2a, b2a, w2b, b2b,
      clssm, mcw, mcb, gaw, gab, bew, beb, fcw, fcb)


def _fold(w, bc, g, be):
    s = g * BN_SCALE
    return (w * s[:, None]).T, (s * bc + be)[None, :]


def kernel(x, clss, mask, params, buffers):
    p = params
    ptsT = jnp.transpose(x, (0, 2, 1))
    px, py, pz = ptsT[..., 0], ptsT[..., 1], ptsT[..., 2]
    featT = jnp.stack(
        [px, py, pz, px * px, px * py, px * pz, py * py, py * pz, pz * pz],
        axis=-1)

    w1, b1 = _fold(p['mlp1_w1'], p['mlp1_b1'], p['mlp1_g1'], p['mlp1_be1'])
    w2, b2 = _fold(p['mlp1_w2'], p['mlp1_b2'], p['mlp1_g2'], p['mlp1_be2'])
    w3, b3 = _fold(p['mlp1_w3'], p['mlp1_b3'], p['mlp1_g3'], p['mlp1_be3'])
    idx, h = _knn_cov_mlp1(ptsT, x, featT, w1, b1, w2, b2, w3, b3)

    col = (idx + (jnp.arange(B, dtype=jnp.int32) * N)[:, None, None]).reshape(-1)
    row = jnp.repeat(jnp.arange(BN, dtype=jnp.int32), K)
    gl_idx = jnp.stack([row, col])

    colg2 = col.reshape(BN * K // 128, 128)
    aggsum = _agg_call(h, colg2)

    wg = p['g1_w'].T * INV_K
    bg = p['g1_b'][None, :]
    pw1, pb1 = p['pe_w1'].T, p['pe_b1'][None, :]
    pw2, pb2 = p['pe_w2'].T, p['pe_b2'][None, :]
    w2a, b2a = _fold(p['mlp2_w1'], p['mlp2_b1'], p['mlp2_g1'], p['mlp2_be1'])
    w2b, b2b = _fold(p['mlp2_w2'], p['mlp2_b2'], p['mlp2_g2'], p['mlp2_be2'])
    peT = jnp.transpose(buffers['pe'][0, :, :N], (1, 0))
    mcw, mcb = _fold(p['mc_w'], p['mc_b'], p['mc_g'], p['mc_be'])
    gaw, gab = _fold(p['gam_w'], p['gam_b'], p['gam_g'], p['gam_be'])
    bew, beb = _fold(p['bet_w'], p['bet_b'], p['bet_g'], p['bet_be'])
    fcw, fcb = p['fc_w'].T, p['fc_b'][None, :]
    clssm = clss[:, :, 0]
    out = _film_mlp2_pool_head(aggsum, peT, wg, bg, pw1, pb1, pw2, pb2,
                               w2a, b2a, w2b, b2b, clssm, mcw, mcb,
                               gaw, gab, bew, beb, fcw, fcb)
    return out, gl_idx


# speedup vs baseline: 1.4205x; 1.2384x over previous
import functools
import math

import jax
import jax.numpy as jnp
from jax import lax
from jax.experimental import pallas as pl
from jax.experimental.pallas import tpu as pltpu
from jax.experimental.pallas import tpu_sc as plsc

B, N, NF, CLS, K = 4, 4096, 128, 16, 16
F2 = NF * 2
BN = B * N
FILM_K = 0.5
BN_SCALE = 1.0 / math.sqrt(1.0 + 1e-5)

NC, NS = 2, 16
NW = NC * NS
NPW = BN // NW

KNN_R = 512


KNN_L = 5
KNN_NBLK = 128
KNN_NPOS = N // KNN_NBLK


def _knn_body(ptsT_ref, x_ref, idx_ref):
    rows = ptsT_ref[0]
    cols = x_ref[0]
    sqc = jnp.sum(cols * cols, axis=0, keepdims=True)
    sqr = jnp.sum(rows * rows, axis=1, keepdims=True)
    dot = jnp.dot(rows, cols, preferred_element_type=jnp.float32)
    d = sqr + sqc - 2.0 * dot
    cur = [d[:, s * KNN_NBLK:(s + 1) * KNN_NBLK] for s in range(KNN_NPOS)]
    lane = lax.broadcasted_iota(jnp.int32, (KNN_R, KNN_NBLK), 1)
    vs, gs = [], []
    for l in range(KNN_L):
        bm = cur[0]
        for s in range(1, KNN_NPOS):
            bm = jnp.minimum(bm, cur[s])
        ap = jnp.full((KNN_R, KNN_NBLK), KNN_NPOS, jnp.int32)
        for s in reversed(range(KNN_NPOS)):
            ap = jnp.where(cur[s] == bm, jnp.int32(s), ap)
        vs.append(bm)
        gs.append(ap * KNN_NBLK + lane)
        if l < KNN_L - 1:
            for s in range(KNN_NPOS):
                cur[s] = jnp.where(ap == jnp.int32(s), jnp.float32(jnp.inf), cur[s])
    bmv, rep = vs[0], gs[0]
    lvl = jnp.zeros((KNN_R, KNN_NBLK), jnp.int32)
    INF = jnp.full((KNN_R, KNN_NBLK), jnp.inf, jnp.float32)
    NEG = jnp.full((KNN_R, KNN_NBLK), -1, jnp.int32)
    outs = []
    for _ in range(K):
        m = jnp.min(bmv, axis=1, keepdims=True)
        a = jnp.min(jnp.where(bmv == m, rep, jnp.int32(N)), axis=1, keepdims=True)
        outs.append(a)
        w = rep == a
        lvl = lvl + w.astype(jnp.int32)
        nv, ng = INF, NEG
        for l in range(KNN_L - 1, 0, -1):
            c = lvl == l
            nv = jnp.where(c, vs[l], nv)
            ng = jnp.where(c, gs[l], ng)
        bmv = jnp.where(w, nv, bmv)
        rep = jnp.where(w, ng, rep)
    idx_ref[0] = jnp.concatenate(outs, axis=1)


def _knn_topk(ptsT, x):
    return pl.pallas_call(
        _knn_body,
        grid=(B, N // KNN_R),
        in_specs=[
            pl.BlockSpec((1, KNN_R, 3), lambda b, t: (b, t, 0)),
            pl.BlockSpec((1, 3, N), lambda b, t: (b, 0, 0)),
        ],
        out_specs=pl.BlockSpec((1, KNN_R, K), lambda b, t: (b, t, 0)),
        out_shape=jax.ShapeDtypeStruct((B, N, K), jnp.int32),
        compiler_params=pltpu.CompilerParams(
            dimension_semantics=("parallel", "parallel")),
    )(ptsT, x)


def _cov_kernel_body(xc, yc, zc, colt, covt, xt, yt, zt, idxb, outb):
    wid = lax.axis_index("s") * NC + lax.axis_index("c")
    base = wid * NPW
    pltpu.sync_copy(xc, xt)
    pltpu.sync_copy(yc, yt)
    pltpu.sync_copy(zc, zt)
    pltpu.sync_copy(colt.at[:, pl.ds(base, NPW)], idxb)

    def body(g, carry):
        off = g * 16
        sx = jnp.zeros((16,), jnp.float32)
        sy = jnp.zeros((16,), jnp.float32)
        sz = jnp.zeros((16,), jnp.float32)
        for j in range(K):
            iv = idxb[j, pl.ds(off, 16)]
            sx = sx + plsc.load_gather(xt, [iv])
            sy = sy + plsc.load_gather(yt, [iv])
            sz = sz + plsc.load_gather(zt, [iv])
        mx, my, mz = sx * 0.0625, sy * 0.0625, sz * 0.0625
        a00 = jnp.zeros((16,), jnp.float32)
        a01 = jnp.zeros((16,), jnp.float32)
        a02 = jnp.zeros((16,), jnp.float32)
        a11 = jnp.zeros((16,), jnp.float32)
        a12 = jnp.zeros((16,), jnp.float32)
        a22 = jnp.zeros((16,), jnp.float32)
        for j in range(K):
            iv = idxb[j, pl.ds(off, 16)]
            cx = plsc.load_gather(xt, [iv]) - mx
            cy = plsc.load_gather(yt, [iv]) - my
            cz = plsc.load_gather(zt, [iv]) - mz
            a00 = a00 + cx * cx
            a01 = a01 + cx * cy
            a02 = a02 + cx * cz
            a11 = a11 + cy * cy
            a12 = a12 + cy * cz
            a22 = a22 + cz * cz
        outb[0, pl.ds(off, 16)] = a00
        outb[1, pl.ds(off, 16)] = a01
        outb[2, pl.ds(off, 16)] = a02
        outb[3, pl.ds(off, 16)] = a01
        outb[4, pl.ds(off, 16)] = a11
        outb[5, pl.ds(off, 16)] = a12
        outb[6, pl.ds(off, 16)] = a02
        outb[7, pl.ds(off, 16)] = a12
        outb[8, pl.ds(off, 16)] = a22
        return carry

    lax.fori_loop(0, NPW // 16, body, 0)
    pltpu.sync_copy(outb, covt.at[:, pl.ds(base, NPW)])


_cov_call = functools.partial(
    pl.kernel,
    out_type=jax.ShapeDtypeStruct((9, BN), jnp.float32),
    mesh=plsc.VectorSubcoreMesh(
        core_axis_name="c", subcore_axis_name="s", num_cores=NC, num_subcores=NS
    ),
    scratch_types=[
        pltpu.VMEM((BN,), jnp.float32),
        pltpu.VMEM((BN,), jnp.float32),
        pltpu.VMEM((BN,), jnp.float32),
        pltpu.VMEM((K, NPW), jnp.int32),
        pltpu.VMEM((9, NPW), jnp.float32),
    ],
    compiler_params=pltpu.CompilerParams(needs_layout_passes=False),
)(_cov_kernel_body)


AGG_CH = 8
AGG_NCHK = NPW // AGG_CH


def _agg_kernel_body(h, colg2, out, idxb, rb0, rb1, outb, sem0, sem1):
    wid = lax.axis_index("s") * NC + lax.axis_index("c")
    base = wid * NPW
    pltpu.sync_copy(colg2.at[pl.ds(wid * AGG_NCHK, AGG_NCHK)], idxb)

    def accum(rb, c):
        for n in range(AGG_CH):
            node = c * AGG_CH + n
            for q in range(4):
                acc = rb[n * K, pl.ds(q * 16, 16)]
                for r in range(1, K):
                    acc = acc + rb[n * K + r, pl.ds(q * 16, 16)]
                outb[node, pl.ds(q * 16, 16)] = acc

    cp0 = pltpu.async_copy(h.at[idxb.at[0]], rb0, sem0)
    cp1 = pltpu.async_copy(h.at[idxb.at[1]], rb1, sem1)

    def body(i, carry):
        c = i * 2
        pltpu.make_async_copy(h.at[idxb.at[c]], rb0, sem0).wait()
        accum(rb0, c)

        @pl.when(c + 2 < AGG_NCHK)
        def _():
            pltpu.async_copy(h.at[idxb.at[c + 2]], rb0, sem0)

        pltpu.make_async_copy(h.at[idxb.at[c + 1]], rb1, sem1).wait()
        accum(rb1, c + 1)

        @pl.when(c + 3 < AGG_NCHK)
        def _():
            pltpu.async_copy(h.at[idxb.at[c + 3]], rb1, sem1)

        return carry

    lax.fori_loop(0, AGG_NCHK // 2, body, 0)
    pltpu.sync_copy(outb, out.at[pl.ds(base, NPW)])


_agg_call = functools.partial(
    pl.kernel,
    out_type=jax.ShapeDtypeStruct((BN, 64), jnp.float32),
    mesh=plsc.VectorSubcoreMesh(
        core_axis_name="c", subcore_axis_name="s", num_cores=NC, num_subcores=NS
    ),
    scratch_types=[
        pltpu.VMEM((AGG_NCHK, AGG_CH * K), jnp.int32),
        pltpu.VMEM((AGG_CH * K, 64), jnp.float32),
        pltpu.VMEM((AGG_CH * K, 64), jnp.float32),
        pltpu.VMEM((NPW, 64), jnp.float32),
        pltpu.SemaphoreType.DMA,
        pltpu.SemaphoreType.DMA,
    ],
    compiler_params=pltpu.CompilerParams(
        needs_layout_passes=False, use_tc_tiling_on_sc=False
    ),
)(_agg_kernel_body)


MLP1_R = 2048


def _mlp1_body(inp_ref, w1, b1, w2, b2, w3, b3, out_ref):
    h = jnp.maximum(
        jnp.dot(inp_ref[...], w1[...], preferred_element_type=jnp.float32) + b1[...], 0.0)
    h = jnp.maximum(
        jnp.dot(h, w2[...], preferred_element_type=jnp.float32) + b2[...], 0.0)
    h = jnp.maximum(
        jnp.dot(h, w3[...], preferred_element_type=jnp.float32) + b3[...], 0.0)
    out_ref[...] = h


def _mlp1(inp, w1, b1, w2, b2, w3, b3):
    full = lambda *s: pl.BlockSpec(s, lambda t: tuple(0 for _ in s))
    return pl.pallas_call(
        _mlp1_body,
        grid=(BN // MLP1_R,),
        in_specs=[
            pl.BlockSpec((MLP1_R, 12), lambda t: (t, 0)),
            full(12, 64), full(1, 64), full(64, 64), full(1, 64),
            full(64, 64), full(1, 64),
        ],
        out_specs=pl.BlockSpec((MLP1_R, 64), lambda t: (t, 0)),
        out_shape=jax.ShapeDtypeStruct((BN, 64), jnp.float32),
    )(inp, w1, b1, w2, b2, w3, b3)


D2_R = 1024
D2_TPB = N // D2_R


def _film_body(agg_ref, pe_ref, wg, bg, pw1, pb1, pw2, pb2,
               w2a, b2a, w2b, b2b, pooled_ref):
    t = pl.program_id(0)
    g = jnp.dot(agg_ref[...], wg[...], preferred_element_type=jnp.float32) + bg[...]
    f1 = jnp.maximum(
        jnp.dot(g, pw1[...], preferred_element_type=jnp.float32) + pb1[...], 0.0)
    film = jnp.dot(f1, pw2[...], preferred_element_type=jnp.float32) + pb2[...]
    gm = film[:, :F2]
    bt = film[:, F2:]
    h2 = gm * pe_ref[...] + bt
    h2 = jnp.maximum(
        jnp.dot(h2, w2a[...], preferred_element_type=jnp.float32) + b2a[...], 0.0)
    h2 = jnp.maximum(
        jnp.dot(h2, w2b[...], preferred_element_type=jnp.float32) + b2b[...], 0.0)
    tmax = jnp.max(h2, axis=0, keepdims=True).reshape(1, 1, F2)

    @pl.when(t % D2_TPB == 0)
    def _():
        pooled_ref[...] = tmax

    @pl.when(t % D2_TPB != 0)
    def _():
        pooled_ref[...] = jnp.maximum(pooled_ref[...], tmax)


def _film_mlp2_pool(aggsum, peT, wg, bg, pw1, pb1, pw2, pb2, w2a, b2a, w2b, b2b):
    full = lambda *s: pl.BlockSpec(s, lambda t: tuple(0 for _ in s))
    return pl.pallas_call(
        _film_body,
        grid=(BN // D2_R,),
        in_specs=[
            pl.BlockSpec((D2_R, 64), lambda t: (t, 0)),
            pl.BlockSpec((D2_R, F2), lambda t: (t % D2_TPB, 0)),
            full(64, F2), full(1, F2),
            full(F2, NF), full(1, NF),
            full(NF, 2 * F2), full(1, 2 * F2),
            full(F2, F2), full(1, F2),
            full(F2, F2), full(1, F2),
        ],
        out_specs=pl.BlockSpec((1, 1, F2), lambda t: (t // D2_TPB, 0, 0)),
        out_shape=jax.ShapeDtypeStruct((B, 1, F2), jnp.float32),
    )(aggsum, peT, wg, bg, pw1, pb1, pw2, pb2, w2a, b2a, w2b, b2b)


def _head_body(pooled_ref, clss_ref, mcw, mcb, gaw, gab, bew, beb, fcw, fcb,
               out_ref):
    ce = jnp.maximum(
        jnp.dot(clss_ref[...], mcw[...], preferred_element_type=jnp.float32)
        + mcb[...], 0.0)
    ga = jnp.maximum(
        jnp.dot(ce, gaw[...], preferred_element_type=jnp.float32) + gab[...], 0.0)
    be = jnp.maximum(
        jnp.dot(ce, bew[...], preferred_element_type=jnp.float32) + beb[...], 0.0)
    xx = pooled_ref[...]
    xx = FILM_K * (ga * xx + be) + (1.0 - FILM_K) * xx
    out = jnp.dot(xx, fcw[...], preferred_element_type=jnp.float32) + fcb[...]
    nrm = jnp.maximum(jnp.sqrt(jnp.sum(out * out, axis=1, keepdims=True)), 1e-12)
    out_ref[...] = out / nrm


def _head(pooled, clssm, mcw, mcb, gaw, gab, bew, beb, fcw, fcb):
    return pl.pallas_call(
        _head_body,
        out_shape=jax.ShapeDtypeStruct((B, NF), jnp.float32),
    )(pooled, clssm, mcw, mcb, gaw, gab, bew, beb, fcw, fcb)


def _fold(w, bc, g, be):
    s = g * BN_SCALE
    return (w * s[:, None]).T, (s * bc + be)[None, :]


def kernel(x, clss, mask, params, buffers):
    p = params
    ptsT = jnp.transpose(x, (0, 2, 1))

    idx = _knn_topk(ptsT, x)

    col = (idx + (jnp.arange(B, dtype=jnp.int32) * N)[:, None, None]).reshape(-1)
    row = jnp.repeat(jnp.arange(BN, dtype=jnp.int32), K)
    gl_idx = jnp.stack([row, col])

    xc = x[:, 0, :].reshape(-1)
    yc = x[:, 1, :].reshape(-1)
    zc = x[:, 2, :].reshape(-1)
    colt = col.reshape(BN, K).T.copy()
    covt = _cov_call(xc, yc, zc, colt)

    w1, b1 = _fold(p['mlp1_w1'], p['mlp1_b1'], p['mlp1_g1'], p['mlp1_be1'])
    w2, b2 = _fold(p['mlp1_w2'], p['mlp1_b2'], p['mlp1_g2'], p['mlp1_be2'])
    w3, b3 = _fold(p['mlp1_w3'], p['mlp1_b3'], p['mlp1_g3'], p['mlp1_be3'])
    inp = jnp.concatenate([ptsT.reshape(BN, 3), covt.T], axis=1)
    h = _mlp1(inp, w1, b1, w2, b2, w3, b3)

    colg2 = col.reshape(BN * K // 128, 128)
    aggsum = _agg_call(h, colg2)

    wg = p['g1_w'].T * (1.0 / K)
    bg = p['g1_b'][None, :]
    pw1, pb1 = p['pe_w1'].T, p['pe_b1'][None, :]
    pw2, pb2 = p['pe_w2'].T, p['pe_b2'][None, :]
    w2a, b2a = _fold(p['mlp2_w1'], p['mlp2_b1'], p['mlp2_g1'], p['mlp2_be1'])
    w2b, b2b = _fold(p['mlp2_w2'], p['mlp2_b2'], p['mlp2_g2'], p['mlp2_be2'])
    peT = jnp.transpose(buffers['pe'][0, :, :N], (1, 0))
    pooled = _film_mlp2_pool(aggsum, peT, wg, bg, pw1, pb1, pw2, pb2,
                             w2a, b2a, w2b, b2b)[:, 0, :]

    mcw, mcb = _fold(p['mc_w'], p['mc_b'], p['mc_g'], p['mc_be'])
    gaw, gab = _fold(p['gam_w'], p['gam_b'], p['gam_g'], p['gam_be'])
    bew, beb = _fold(p['bet_w'], p['bet_b'], p['bet_g'], p['bet_be'])
    fcw, fcb = p['fc_w'].T, p['fc_b'][None, :]
    clssm = clss[:, :, 0]
    out = _head(pooled, clssm, mcw, mcb, gaw, gab, bew, beb, fcw, fcb)

    return out, gl_idx
